# baseline (speedup 1.0000x reference)
import functools
import math

import jax
import jax.numpy as jnp
from jax import lax
from jax.experimental import pallas as pl
from jax.experimental.pallas import tpu as pltpu
from jax.experimental.pallas import tpu_sc as plsc

D_MODEL = 32
SCALE = math.sqrt(D_MODEL)

NC = 2
NS = 16
NW = NC * NS

CHUNK = 128
NBUF = 8

TBLK = 8192


def _tc_relayout(lut_t):
    vocab = lut_t.shape[1]
    grid = (vocab + TBLK - 1) // TBLK

    def body(l_ref, e_ref, o_ref):
        o_ref[...] = lax.dot_general(
            l_ref[...], e_ref[...], (((0,), (0,)), ((), ())),
            preferred_element_type=jnp.float32,
        )

    return pl.pallas_call(
        body,
        grid=(grid,),
        in_specs=[
            pl.BlockSpec((D_MODEL, TBLK), lambda i: (0, i)),
            pl.BlockSpec((D_MODEL, D_MODEL), lambda i: (0, 0)),
        ],
        out_specs=pl.BlockSpec((TBLK, D_MODEL), lambda i: (i, 0)),
        out_shape=jax.ShapeDtypeStruct((vocab, D_MODEL), jnp.float32),
    )(lut_t, jnp.eye(D_MODEL, dtype=jnp.float32) * SCALE)


def _tc_outperm(f4, n_j):
    def body(x_ref, e_ref, o_ref):
        x3 = x_ref[0]
        for cb in range(4):
            o_ref[0, cb] = lax.dot_general(
                x3[:, :, cb * 8:(cb + 1) * 8], e_ref[...],
                (((1,), (0,)), ((), ())),
                preferred_element_type=jnp.float32,
            )

    return pl.pallas_call(
        body,
        grid=(n_j,),
        in_specs=[
            pl.BlockSpec((1, NW, CHUNK, D_MODEL), lambda j: (j, 0, 0, 0)),
            pl.BlockSpec((CHUNK, CHUNK), lambda j: (0, 0)),
        ],
        out_specs=pl.BlockSpec(
            (1, 4, NW, 8, CHUNK), lambda j: (j, 0, 0, 0, 0)
        ),
        out_shape=jax.ShapeDtypeStruct((n_j, 4, NW, 8, CHUNK), jnp.float32),
    )(f4, jnp.eye(CHUNK, dtype=jnp.float32))


def _make_sc_gather(n_j):
    out_shape = (n_j, NW, CHUNK, D_MODEL)

    @functools.partial(
        pl.kernel,
        out_type=jax.ShapeDtypeStruct(out_shape, jnp.float32),
        mesh=plsc.VectorSubcoreMesh(core_axis_name="c", subcore_axis_name="s"),
        scratch_types=[
            pltpu.VMEM((n_j, CHUNK), jnp.int32),
            pltpu.VMEM((NBUF, CHUNK, D_MODEL), jnp.float32),
            pltpu.VMEM((NBUF, CHUNK, D_MODEL), jnp.float32),
        ]
        + [pltpu.SemaphoreType.DMA] * (2 * NBUF),
        compiler_params=pltpu.CompilerParams(
            use_tc_tiling_on_sc=False, needs_layout_passes=False
        ),
    )
    def body(idx_hbm, table_hbm, out_hbm, idx_v, gbuf, obuf, *sems):
        gsems = sems[:NBUF]
        osems = sems[NBUF:]
        c = lax.axis_index("c")
        s = lax.axis_index("s")
        wid = s * NC + c
        pltpu.sync_copy(idx_hbm.at[wid], idx_v)

        def issue_gather(g, b):
            pltpu.async_copy(table_hbm.at[idx_v.at[g]], gbuf.at[b], gsems[b])

        def wait_gather(g, b):
            pltpu.make_async_copy(
                table_hbm.at[idx_v.at[g]], gbuf.at[b], gsems[b]
            ).wait()

        def issue_out(g, b):
            pltpu.async_copy(obuf.at[b], out_hbm.at[g, wid], osems[b])

        def wait_out(g, b):
            pltpu.make_async_copy(
                obuf.at[b], out_hbm.at[g, wid], osems[b]
            ).wait()

        def copy_chunk(b):
            def row_body(ic, carry):
                obuf[b, ic, 0:16] = gbuf[b, ic, 0:16]
                obuf[b, ic, 16:32] = gbuf[b, ic, 16:32]
                return carry

            lax.fori_loop(0, CHUNK, row_body, 0, unroll=8)

        for b in range(NBUF):
            issue_gather(b, b)

        niter = n_j // NBUF

        def mid(i, carry):
            for b in range(NBUF):
                g = i * NBUF + b
                wait_gather(g, b)

                @pl.when(g >= NBUF)
                def _():
                    wait_out(g - NBUF, b)

                copy_chunk(b)

                @pl.when(g + NBUF < n_j)
                def _():
                    issue_gather(g + NBUF, b)

                issue_out(g, b)
            return carry

        lax.fori_loop(0, niter, mid, 0)

        for b in range(NBUF):
            wait_out((niter - 1) * NBUF + b, b)

    return body


def kernel(x, lut):
    n_i, n_j = x.shape
    table = _tc_relayout(lut.T)
    xi = jnp.transpose(
        jnp.asarray(x, jnp.int32).T.reshape(n_j, NW, CHUNK), (1, 0, 2)
    )
    f4 = _make_sc_gather(n_j)(xi, table)
    out5 = _tc_outperm(f4, n_j)
    return out5.transpose(2, 4, 0, 1, 3).reshape(n_i, n_j, D_MODEL)


# speedup vs baseline: 1.3167x; 1.3167x over previous
import functools
import math

import jax
import jax.numpy as jnp
from jax import lax
from jax.experimental import pallas as pl
from jax.experimental.pallas import tpu as pltpu
from jax.experimental.pallas import tpu_sc as plsc

D_MODEL = 32
SCALE = math.sqrt(D_MODEL)

NC = 2
NS = 16
NW = NC * NS

CHUNK = 128
NBUF = 8


def _make_kernel(total_rows):
    chunks_per_w = total_rows // (NW * CHUNK)
    rows_per_w = chunks_per_w * CHUNK
    niter = chunks_per_w // NBUF

    @functools.partial(
        pl.kernel,
        out_type=jax.ShapeDtypeStruct((total_rows, D_MODEL), jnp.float32),
        mesh=plsc.VectorSubcoreMesh(core_axis_name="c", subcore_axis_name="s"),
        scratch_types=[
            pltpu.VMEM((chunks_per_w, CHUNK), jnp.int32),
            pltpu.VMEM((NBUF, CHUNK, D_MODEL), jnp.float32),
            pltpu.VMEM((NBUF, CHUNK, D_MODEL), jnp.float32),
        ]
        + [pltpu.SemaphoreType.DMA] * (2 * NBUF),
        compiler_params=pltpu.CompilerParams(use_tc_tiling_on_sc=False),
    )
    def body(idx_hbm, table_hbm, out_hbm, idx_v, gbuf, obuf, *sems):
        gsems = sems[:NBUF]
        osems = sems[NBUF:]
        c = lax.axis_index("c")
        s = lax.axis_index("s")
        wid = s * NC + c
        base = wid * rows_per_w
        pltpu.sync_copy(idx_hbm.at[wid], idx_v)

        def issue_gather(g, b):
            pltpu.async_copy(table_hbm.at[idx_v.at[g]], gbuf.at[b], gsems[b])

        def wait_gather(g, b):
            pltpu.make_async_copy(
                table_hbm.at[idx_v.at[g]], gbuf.at[b], gsems[b]
            ).wait()

        def issue_out(g, b):
            pltpu.async_copy(
                obuf.at[b], out_hbm.at[pl.ds(base + g * CHUNK, CHUNK)], osems[b]
            )

        def wait_out(g, b):
            pltpu.make_async_copy(
                obuf.at[b], out_hbm.at[pl.ds(base + g * CHUNK, CHUNK)], osems[b]
            ).wait()

        def scale(b):
            def row_body(r, rc):
                obuf[b, r, 0:16] = gbuf[b, r, 0:16] * SCALE
                obuf[b, r, 16:32] = gbuf[b, r, 16:32] * SCALE
                return rc

            lax.fori_loop(0, CHUNK, row_body, 0, unroll=8)

        for b in range(NBUF):
            issue_gather(b, b)

        for b in range(NBUF):
            wait_gather(b, b)
            scale(b)
            issue_gather(b + NBUF, b)
            issue_out(b, b)

        def mid(i, carry):
            for b in range(NBUF):
                g = i * NBUF + b
                wait_gather(g, b)
                wait_out(g - NBUF, b)
                scale(b)
                issue_gather(g + NBUF, b)
                issue_out(g, b)
            return carry

        lax.fori_loop(1, niter - 1, mid, 0)

        for b in range(NBUF):
            g = (niter - 1) * NBUF + b
            wait_gather(g, b)
            wait_out(g - NBUF, b)
            scale(b)
            issue_out(g, b)
        for b in range(NBUF):
            wait_out((niter - 1) * NBUF + b, b)

    return body


def kernel(x, lut):
    total = x.shape[0] * x.shape[1]
    chunks_per_w = total // (NW * CHUNK)
    xi = jnp.asarray(x, jnp.int32).reshape(NW, chunks_per_w, CHUNK)
    out = _make_kernel(total)(xi, lut)
    return out.reshape(x.shape[0], x.shape[1], D_MODEL)
